# baseline (speedup 1.0000x reference)
import jax
import jax.numpy as jnp
from jax.experimental import pallas as pl

_TOPK = 8


def _fused_body(emb_ref, mask_ref, w_ref, b_ref, pooled_ref, attn_ref):
    e = emb_ref[0]
    w = w_ref[...]
    K = e.shape[0]
    e16 = e.astype(jnp.bfloat16).astype(jnp.float32)
    w16 = w.astype(jnp.bfloat16).astype(jnp.float32)
    s = jnp.sum(e16 * w16, axis=1)
    s = s.reshape(1, K) + b_ref[0, 0]
    m = mask_ref[0]
    s = jnp.where(m == 0.0, -jnp.inf, s)

    iota = jax.lax.broadcasted_iota(jnp.int32, (1, K), 1)
    s_work = jnp.maximum(s, jnp.float32(-3.0e38))
    attn = jnp.zeros((1, K), dtype=jnp.float32)
    pooled = jnp.zeros((1, e.shape[1]), dtype=jnp.float32)
    inv_k = jnp.float32(1.0 / _TOPK)
    for _ in range(_TOPK):
        v = jnp.max(s_work, axis=1, keepdims=True)
        cand = jnp.where(s_work == v, iota, K)
        idx = jnp.min(cand, axis=1, keepdims=True)
        sel = iota == idx
        attn = attn + jnp.where(sel, inv_k, 0.0)
        s_work = jnp.where(sel, -jnp.inf, s_work)
        row = emb_ref[0, pl.ds(idx[0, 0], 1), :]
        pooled = pooled + row * inv_k
    pooled_ref[0] = pooled
    attn_ref[0] = attn


def kernel(embeddings, mask, W, b):
    B, K, D = embeddings.shape
    b2 = b.reshape(1, 1)
    mask3 = mask.reshape(B, 1, K)
    pooled, attn = pl.pallas_call(
        _fused_body,
        grid=(B,),
        in_specs=[
            pl.BlockSpec((1, K, D), lambda i: (i, 0, 0)),
            pl.BlockSpec((1, 1, K), lambda i: (i, 0, 0)),
            pl.BlockSpec((1, D), lambda i: (0, 0)),
            pl.BlockSpec((1, 1), lambda i: (0, 0)),
        ],
        out_specs=[
            pl.BlockSpec((1, 1, D), lambda i: (i, 0, 0)),
            pl.BlockSpec((1, 1, K), lambda i: (i, 0, 0)),
        ],
        out_shape=[
            jax.ShapeDtypeStruct((B, 1, D), jnp.float32),
            jax.ShapeDtypeStruct((B, 1, K), jnp.float32),
        ],
    )(embeddings, mask3, W, b2)
    return (pooled.reshape(B, D), attn.reshape(B, K))


# speedup vs baseline: 1.1744x; 1.1744x over previous
import jax
import jax.numpy as jnp
from jax.experimental import pallas as pl

_TOPK = 8
_ROWS = 32


def _fused_body(emb_ref, mask_ref, w_ref, b_ref, pooled_ref, attn_ref):
    e = emb_ref[0]
    K, D = e.shape
    C = K // _ROWS
    e16 = e.astype(jnp.bfloat16)
    w16 = w_ref[...].astype(jnp.bfloat16)
    s = jax.lax.dot_general(
        e16, w16,
        dimension_numbers=(((1,), (0,)), ((), ())),
        preferred_element_type=jnp.float32,
    )
    s = s.reshape(_ROWS, C) + b_ref[0, 0]
    m = mask_ref[0]
    s = jnp.where(m == 0.0, -jnp.inf, s)

    row_i = jax.lax.broadcasted_iota(jnp.int32, (_ROWS, C), 0)
    col_i = jax.lax.broadcasted_iota(jnp.int32, (_ROWS, C), 1)
    gidx = row_i * C + col_i
    s_work = jnp.maximum(s, jnp.float32(-3.0e38))
    attn = jnp.zeros((_ROWS, C), dtype=jnp.float32)
    pooled = jnp.zeros((1, D), dtype=jnp.float32)
    inv_k = jnp.float32(1.0 / _TOPK)
    for _ in range(_TOPK):
        v = jnp.max(s_work)
        cand = jnp.where(s_work == v, gidx, K)
        idx = jnp.min(cand)
        sel = gidx == idx
        attn = attn + jnp.where(sel, inv_k, 0.0)
        s_work = jnp.where(sel, -jnp.inf, s_work)
        row = emb_ref[0, pl.ds(idx, 1), :]
        pooled = pooled + row * inv_k
    pooled_ref[0] = pooled
    attn_ref[0] = attn


def kernel(embeddings, mask, W, b):
    B, K, D = embeddings.shape
    C = K // _ROWS
    b2 = b.reshape(1, 1)
    w_t = W.reshape(D, 1)
    mask4 = mask.reshape(B, _ROWS, C)
    pooled, attn = pl.pallas_call(
        _fused_body,
        grid=(B,),
        in_specs=[
            pl.BlockSpec((1, K, D), lambda i: (i, 0, 0)),
            pl.BlockSpec((1, _ROWS, C), lambda i: (i, 0, 0)),
            pl.BlockSpec((D, 1), lambda i: (0, 0)),
            pl.BlockSpec((1, 1), lambda i: (0, 0)),
        ],
        out_specs=[
            pl.BlockSpec((1, 1, D), lambda i: (i, 0, 0)),
            pl.BlockSpec((1, _ROWS, C), lambda i: (i, 0, 0)),
        ],
        out_shape=[
            jax.ShapeDtypeStruct((B, 1, D), jnp.float32),
            jax.ShapeDtypeStruct((B, _ROWS, C), jnp.float32),
        ],
    )(embeddings, mask4, w_t, b2)
    return (pooled.reshape(B, D), attn.reshape(B, K))


# speedup vs baseline: 1.1768x; 1.0020x over previous
import jax
import jax.numpy as jnp
from jax.experimental import pallas as pl

_TOPK = 8
_ROWS = 32


def _fused_body(emb_ref, mask_ref, w_ref, b_ref, pooled_ref, attn_ref):
    e = emb_ref[0]
    K, D = e.shape
    C = K // _ROWS
    s = jax.lax.dot_general(
        e, w_ref[...],
        dimension_numbers=(((1,), (0,)), ((), ())),
        precision=jax.lax.Precision.DEFAULT,
        preferred_element_type=jnp.float32,
    )
    s = s.reshape(_ROWS, C) + b_ref[0, 0]
    m = mask_ref[0]
    s = jnp.where(m == 0.0, -jnp.inf, s)

    row_i = jax.lax.broadcasted_iota(jnp.int32, (_ROWS, C), 0)
    col_i = jax.lax.broadcasted_iota(jnp.int32, (_ROWS, C), 1)
    gidx = row_i * C + col_i
    s_work = jnp.maximum(s, jnp.float32(-3.0e38))
    attn = jnp.zeros((_ROWS, C), dtype=jnp.float32)
    pooled = jnp.zeros((1, D), dtype=jnp.float32)
    inv_k = jnp.float32(1.0 / _TOPK)
    for _ in range(_TOPK):
        v = jnp.max(s_work)
        cand = jnp.where(s_work == v, gidx, K)
        idx = jnp.min(cand)
        sel = gidx == idx
        attn = attn + jnp.where(sel, inv_k, 0.0)
        s_work = jnp.where(sel, -jnp.inf, s_work)
        row = emb_ref[0, pl.ds(idx, 1), :]
        pooled = pooled + row * inv_k
    pooled_ref[0] = pooled
    attn_ref[0] = attn


def kernel(embeddings, mask, W, b):
    B, K, D = embeddings.shape
    C = K // _ROWS
    b2 = b.reshape(1, 1)
    w_t = W.reshape(D, 1)
    mask4 = mask.reshape(B, _ROWS, C)
    pooled, attn = pl.pallas_call(
        _fused_body,
        grid=(B,),
        in_specs=[
            pl.BlockSpec((1, K, D), lambda i: (i, 0, 0)),
            pl.BlockSpec((1, _ROWS, C), lambda i: (i, 0, 0)),
            pl.BlockSpec((D, 1), lambda i: (0, 0)),
            pl.BlockSpec((1, 1), lambda i: (0, 0)),
        ],
        out_specs=[
            pl.BlockSpec((1, 1, D), lambda i: (i, 0, 0)),
            pl.BlockSpec((1, _ROWS, C), lambda i: (i, 0, 0)),
        ],
        out_shape=[
            jax.ShapeDtypeStruct((B, 1, D), jnp.float32),
            jax.ShapeDtypeStruct((B, _ROWS, C), jnp.float32),
        ],
    )(embeddings, mask4, w_t, b2)
    return (pooled.reshape(B, D), attn.reshape(B, K))
